# baseline (speedup 1.0000x reference)
import jax
import jax.numpy as jnp
from jax import lax
from jax.experimental import pallas as pl
from jax.experimental.pallas import tpu as pltpu
from jax.experimental.pallas import tpu_sc as plsc

N = 10000
E = 320000
D = 128
LN_EPS = 1e-5

NC = 2
NS = 16
NW = NC * NS
EPW = E // NW
K = 80
CH = EPW // K
RPT = 640
RPT_LAST = N - RPT * (NS - 1)
ZR = 80


NSLOT = 3


def _sc_propagate(x_hbm, src_hbm, dst_hbm, out_hbm,
                  acc, sidx, dbufs, rows, gsems, dsems):
    c = lax.axis_index("c")
    s = lax.axis_index("s")
    wid = s * NC + c

    off = pl.multiple_of(wid * EPW, 8)
    d_si = pltpu.async_copy(src_hbm.at[pl.ds(off, EPW)], sidx, gsems[0])

    def gidx(j, h):
        return sidx.at[pl.ds(pl.multiple_of(j * K + h * (K // 2), 8), K // 2)]

    def didx_src(j):
        return dst_hbm.at[pl.ds(pl.multiple_of(wid * EPW + j * K, 8), K)]

    zv = jnp.zeros((16,), jnp.float32)

    def zfill_body(r, carry):
        for q in range(D // 16):
            rows[0][r, pl.ds(q * 16, 16)] = zv
        return carry

    lax.fori_loop(0, K, zfill_body, 0)

    n_zero = jnp.where(s < NS - 1, RPT // ZR, RPT_LAST // ZR)

    def zero_body(i, carry):
        pltpu.sync_copy(rows[0].at[pl.ds(0, ZR)],
                        acc.at[pl.ds(s * RPT + i * ZR, ZR)])
        return carry

    lax.fori_loop(0, n_zero, zero_body, 0)
    d_si.wait()

    def gather_issue(j, b):
        for h in range(2):
            pltpu.async_copy(x_hbm.at[gidx(j, h)],
                             rows[b].at[pl.ds(h * (K // 2), K // 2)],
                             gsems[b])

    def gather_wait(j, b):
        for h in range(2):
            pltpu.make_async_copy(x_hbm.at[gidx(j, h)],
                                  rows[b].at[pl.ds(h * (K // 2), K // 2)],
                                  gsems[b]).wait()

    for b in range(NSLOT):
        pltpu.async_copy(didx_src(b), dbufs[b], dsems[b])
        gather_issue(b, b)
    plsc.subcore_barrier()

    def edge_body(j, carry):
        for b in range(NSLOT):
            @pl.when(j % NSLOT == b)
            def _():
                pltpu.make_async_copy(didx_src(j), dbufs[b], dsems[b]).wait()
                gather_wait(j, b)
                pltpu.sync_copy(rows[b], acc.at[dbufs[b]], add=True)

                @pl.when(j + NSLOT < CH)
                def _():
                    pltpu.async_copy(didx_src(j + NSLOT), dbufs[b], dsems[b])
                    gather_issue(j + NSLOT, b)
        return carry

    lax.fori_loop(0, CH, edge_body, 0)
    plsc.subcore_barrier()

    @pl.when(s < NS - 1)
    def _():
        pltpu.sync_copy(acc.at[pl.ds(s * RPT, RPT)],
                        out_hbm.at[c, pl.ds(s * RPT, RPT)])

    @pl.when(s == NS - 1)
    def _():
        pltpu.sync_copy(acc.at[pl.ds((NS - 1) * RPT, RPT_LAST)],
                        out_hbm.at[c, pl.ds((NS - 1) * RPT, RPT_LAST)])


BLK = 5000


def _loop_msg_body(x_ref, w_ref, o_ref):
    o_ref[...] = jnp.dot(x_ref[...], w_ref[...],
                         preferred_element_type=jnp.float32)


def _combine_body(lm_ref, p_ref, o_ref):
    h = p_ref[0] + p_ref[1] + lm_ref[...]
    mu = jnp.mean(h, axis=-1, keepdims=True)
    var = jnp.mean((h - mu) ** 2, axis=-1, keepdims=True)
    o_ref[...] = (h - mu) * lax.rsqrt(var + LN_EPS)


def kernel(x, edge_index, loop_weight):
    src = edge_index[0]
    dst = edge_index[1]

    propagate = pl.kernel(
        _sc_propagate,
        out_type=jax.ShapeDtypeStruct((NC, N, D), jnp.float32),
        mesh=plsc.VectorSubcoreMesh(core_axis_name="c", subcore_axis_name="s"),
        scratch_types=[
            pltpu.VMEM_SHARED((N, D), jnp.float32),
            pltpu.VMEM((EPW,), jnp.int32),
            [pltpu.VMEM((K,), jnp.int32)] * NSLOT,
            [pltpu.VMEM((K, D), jnp.float32)] * NSLOT,
            [pltpu.SemaphoreType.DMA] * NSLOT,
            [pltpu.SemaphoreType.DMA] * NSLOT,
        ],
    )
    partials = propagate(x, src, dst)

    loop_msg = pl.pallas_call(
        _loop_msg_body,
        out_shape=jax.ShapeDtypeStruct((N, D), jnp.float32),
        grid=(N // BLK,),
        in_specs=[
            pl.BlockSpec((BLK, D), lambda i: (i, 0)),
            pl.BlockSpec((D, D), lambda i: (0, 0)),
        ],
        out_specs=pl.BlockSpec((BLK, D), lambda i: (i, 0)),
    )(x, loop_weight)

    out = pl.pallas_call(
        _combine_body,
        out_shape=jax.ShapeDtypeStruct((N, D), jnp.float32),
        grid=(N // BLK,),
        in_specs=[
            pl.BlockSpec((BLK, D), lambda i: (i, 0)),
            pl.BlockSpec((NC, BLK, D), lambda i: (0, i, 0)),
        ],
        out_specs=pl.BlockSpec((BLK, D), lambda i: (i, 0)),
    )(loop_msg, partials)
    return out


# speedup vs baseline: 1.0048x; 1.0048x over previous
import jax
import jax.numpy as jnp
from jax import lax
from jax.experimental import pallas as pl
from jax.experimental.pallas import tpu as pltpu
from jax.experimental.pallas import tpu_sc as plsc

N = 10000
E = 320000
D = 128
LN_EPS = 1e-5

NC = 2
NS = 16
NW = NC * NS
EPW = E // NW
K = 80
CH = EPW // K
RPT = 640
RPT_LAST = N - RPT * (NS - 1)
ZR = 80


NSLOT = 3


def _sc_propagate(x_hbm, src_hbm, dst_hbm, out_hbm,
                  acc, sidx, dbufs, rows, gsems, dsems, ssems):
    c = lax.axis_index("c")
    s = lax.axis_index("s")
    wid = s * NC + c

    off = pl.multiple_of(wid * EPW, 8)
    d_si = pltpu.async_copy(src_hbm.at[pl.ds(off, EPW)], sidx, gsems[0])

    def gidx(j, h):
        return sidx.at[pl.ds(pl.multiple_of(j * K + h * (K // 2), 8), K // 2)]

    def didx_src(j, h):
        return dst_hbm.at[
            pl.ds(pl.multiple_of(wid * EPW + j * K + h * (K // 2), 8), K // 2)]

    def didx_issue(j, b):
        for h in range(2):
            pltpu.async_copy(didx_src(j, h), dbufs[b].at[8 * h], dsems[b])

    def didx_wait(j, b):
        for h in range(2):
            pltpu.make_async_copy(didx_src(j, h), dbufs[b].at[8 * h],
                                  dsems[b]).wait()

    zv = jnp.zeros((16,), jnp.float32)

    def zfill_body(r, carry):
        for q in range(D // 16):
            rows[0][r, pl.ds(q * 16, 16)] = zv
        return carry

    lax.fori_loop(0, K, zfill_body, 0)

    n_zero = jnp.where(s < NS - 1, RPT // ZR, RPT_LAST // ZR)

    def zero_body(i, carry):
        pltpu.sync_copy(rows[0].at[pl.ds(0, ZR)],
                        acc.at[pl.ds(s * RPT + i * ZR, ZR)])
        return carry

    lax.fori_loop(0, n_zero, zero_body, 0)
    d_si.wait()

    def gather_issue(j, b):
        for h in range(2):
            pltpu.async_copy(x_hbm.at[gidx(j, h)],
                             rows[b].at[pl.ds(h * (K // 2), K // 2)],
                             gsems[b])

    def gather_wait(j, b):
        for h in range(2):
            pltpu.make_async_copy(x_hbm.at[gidx(j, h)],
                                  rows[b].at[pl.ds(h * (K // 2), K // 2)],
                                  gsems[b]).wait()

    def scatter(j, b):
        for h in range(2):
            pltpu.async_copy(rows[b].at[pl.ds(h * (K // 2), K // 2)],
                             acc.at[dbufs[b].at[8 * h]], ssems[b], add=True)
        for h in range(2):
            pltpu.make_async_copy(rows[b].at[pl.ds(h * (K // 2), K // 2)],
                                  acc.at[dbufs[b].at[8 * h]], ssems[b]).wait()

    for b in range(NSLOT):
        didx_issue(b, b)
        gather_issue(b, b)
    plsc.subcore_barrier()

    def edge_body(j, carry):
        for b in range(NSLOT):
            @pl.when(j % NSLOT == b)
            def _():
                didx_wait(j, b)
                gather_wait(j, b)
                scatter(j, b)

                @pl.when(j + NSLOT < CH)
                def _():
                    didx_issue(j + NSLOT, b)
                    gather_issue(j + NSLOT, b)
        return carry

    lax.fori_loop(0, CH, edge_body, 0)
    plsc.subcore_barrier()

    @pl.when(s < NS - 1)
    def _():
        pltpu.sync_copy(acc.at[pl.ds(s * RPT, RPT)],
                        out_hbm.at[c, pl.ds(s * RPT, RPT)])

    @pl.when(s == NS - 1)
    def _():
        pltpu.sync_copy(acc.at[pl.ds((NS - 1) * RPT, RPT_LAST)],
                        out_hbm.at[c, pl.ds((NS - 1) * RPT, RPT_LAST)])


BLK = 5000


def _combine_body(x_ref, w_ref, p_ref, o_ref):
    h = p_ref[0] + p_ref[1] + jnp.dot(x_ref[...], w_ref[...],
                                      preferred_element_type=jnp.float32)
    mu = jnp.mean(h, axis=-1, keepdims=True)
    var = jnp.mean((h - mu) ** 2, axis=-1, keepdims=True)
    o_ref[...] = (h - mu) * lax.rsqrt(var + LN_EPS)


def kernel(x, edge_index, loop_weight):
    src = edge_index[0]
    dst = edge_index[1]

    propagate = pl.kernel(
        _sc_propagate,
        out_type=jax.ShapeDtypeStruct((NC, N, D), jnp.float32),
        mesh=plsc.VectorSubcoreMesh(core_axis_name="c", subcore_axis_name="s"),
        scratch_types=[
            pltpu.VMEM_SHARED((N, D), jnp.float32),
            pltpu.VMEM((EPW,), jnp.int32),
            [pltpu.VMEM((16, K // 2), jnp.int32)] * NSLOT,
            [pltpu.VMEM((K, D), jnp.float32)] * NSLOT,
            [pltpu.SemaphoreType.DMA] * NSLOT,
            [pltpu.SemaphoreType.DMA] * NSLOT,
            [pltpu.SemaphoreType.DMA] * NSLOT,
        ],
    )
    partials = propagate(x, src, dst)

    out = pl.pallas_call(
        _combine_body,
        out_shape=jax.ShapeDtypeStruct((N, D), jnp.float32),
        grid=(N // BLK,),
        in_specs=[
            pl.BlockSpec((BLK, D), lambda i: (i, 0)),
            pl.BlockSpec((D, D), lambda i: (0, 0)),
            pl.BlockSpec((NC, BLK, D), lambda i: (0, i, 0)),
        ],
        out_specs=pl.BlockSpec((BLK, D), lambda i: (i, 0)),
    )(x, loop_weight, partials)
    return out
